# Initial kernel scaffold; baseline (speedup 1.0000x reference)
import jax
import jax.numpy as jnp
from jax.experimental import pallas as pl


def kernel(node, edge, edge_feature, W, a):
    raise NotImplementedError("write your pallas kernel here")


# speedup vs baseline: 5.0758x; 5.0758x over previous
import dataclasses
import functools

import jax
import jax.numpy as jnp
from jax import lax
from jax.experimental import pallas as pl
from jax.experimental.pallas import tpu as pltpu
from jax.experimental.pallas import tpu_sc as plsc

N = 10000
E = 320000
D = 128
ALPHA = 0.2

NC = 2
NS = 16
NW = NC * NS
EPW = E // NW
K = 80
NCHUNK = EPW // K
STRIPE = 624
TAIL = N - NS * STRIPE

RB = 1000
NRB = N // RB


def _mm_body(node_ref, w_ref, a8_ref, h_ref, s8_ref):
    hblk = jnp.dot(node_ref[...], w_ref[...],
                   preferred_element_type=jnp.float32,
                   precision=lax.Precision.HIGHEST)
    h_ref[...] = hblk
    s8_ref[0] = jnp.dot(hblk, a8_ref[...],
                        preferred_element_type=jnp.float32,
                        precision=lax.Precision.HIGHEST)


def _matmul(node, W, a8):
    return pl.pallas_call(
        _mm_body,
        grid=(NRB,),
        in_specs=[
            pl.BlockSpec((RB, D), lambda i: (i, 0)),
            pl.BlockSpec((D, D), lambda i: (0, 0)),
            pl.BlockSpec((D, 8), lambda i: (0, 0)),
        ],
        out_specs=[
            pl.BlockSpec((RB, D), lambda i: (i, 0)),
            pl.BlockSpec((1, RB, 8), lambda i: (i, 0, 0)),
        ],
        out_shape=[
            jax.ShapeDtypeStruct((N, D), jnp.float32),
            jax.ShapeDtypeStruct((NRB, RB, 8), jnp.float32),
        ],
    )(node, W, a8)


_sc_mesh = plsc.VectorSubcoreMesh(core_axis_name="c", subcore_axis_name="s")

_sc_params = pltpu.CompilerParams()
if "needs_layout_passes" in pltpu.CompilerParams.__dataclass_fields__:
    _sc_params = dataclasses.replace(_sc_params, needs_layout_passes=False)


@functools.partial(
    pl.kernel,
    mesh=_sc_mesh,
    compiler_params=_sc_params,
    out_type=[
        jax.ShapeDtypeStruct((E,), jnp.float32),
        jax.ShapeDtypeStruct((NC, N, D), jnp.float32),
        jax.ShapeDtypeStruct((NW, N), jnp.float32),
    ],
    scratch_types=[
        pltpu.VMEM((N,), jnp.float32),
        pltpu.VMEM((N,), jnp.float32),
        pltpu.VMEM((N,), jnp.float32),
        pltpu.VMEM((K,), jnp.int32),
        pltpu.VMEM((K,), jnp.int32),
        pltpu.VMEM((K,), jnp.float32),
        pltpu.VMEM((K,), jnp.float32),
        pltpu.VMEM((K, D), jnp.float32),
        pltpu.VMEM((16,), jnp.float32),
        pltpu.VMEM_SHARED((N, D), jnp.float32),
        pltpu.SemaphoreType.DMA,
    ],
)
def _edge_kernel(h_hbm, src_hbm, dst_hbm, ef_hbm, s1_hbm, s2_hbm, a3_hbm,
                 zfull_hbm, z1d_hbm,
                 ee_hbm, accp_hbm, rsp_hbm,
                 s1_v, s2_v, rs_v, src_v, dst_v, ef_v, ee_v, rows_v, a3_v,
                 acc_S, sem):
    c = lax.axis_index("c")
    s = lax.axis_index("s")
    wid = c * NS + s

    sbase = pl.multiple_of(s * STRIPE, 8)
    pltpu.sync_copy(zfull_hbm, acc_S.at[pl.ds(sbase, STRIPE)])

    @pl.when(s == 0)
    def _tail_init():
        pltpu.sync_copy(zfull_hbm.at[pl.ds(0, TAIL)],
                        acc_S.at[pl.ds(NS * STRIPE, TAIL)])
    pltpu.sync_copy(z1d_hbm, rs_v)
    pltpu.sync_copy(s1_hbm, s1_v)
    pltpu.sync_copy(s2_hbm, s2_v)
    pltpu.sync_copy(a3_hbm, a3_v)
    plsc.subcore_barrier()

    a3 = a3_v[...]
    base0 = wid * EPW

    @pl.loop(0, NCHUNK)
    def _chunk(ci):
        base = base0 + ci * K
        pltpu.sync_copy(src_hbm.at[pl.ds(base, K)], src_v)
        pltpu.sync_copy(dst_hbm.at[pl.ds(base, K)], dst_v)
        pltpu.sync_copy(ef_hbm.at[pl.ds(base, K)], ef_v)
        pltpu.async_copy(h_hbm.at[dst_v], rows_v, sem).wait()

        @pl.loop(0, K, step=16)
        def _grp(g):
            src16 = src_v[pl.ds(g, 16)]
            dst16 = dst_v[pl.ds(g, 16)]
            ef16 = ef_v[pl.ds(g, 16)]
            v1 = plsc.load_gather(s1_v, [src16])
            v2 = plsc.load_gather(s2_v, [dst16])
            lg = v1 + v2 + ef16 * a3
            lr = jnp.where(lg >= 0, lg, ALPHA * lg)
            ee16 = jnp.exp(-lr)
            ee_v[pl.ds(g, 16)] = ee16
            plsc.addupdate_scatter(rs_v, [src16], ee16)

        @pl.loop(0, K)
        def _edge(e):
            idx16 = lax.broadcast(e, (16,))
            spl = plsc.load_gather(ee_v, [idx16])
            for j in range(D // 16):
                rows_v[e, pl.ds(j * 16, 16)] = rows_v[e, pl.ds(j * 16, 16)] * spl

        pltpu.sync_copy(rows_v, acc_S.at[src_v], add=True)
        pltpu.sync_copy(ee_v, ee_hbm.at[pl.ds(base, K)])

    pltpu.sync_copy(rs_v, rsp_hbm.at[wid])
    plsc.subcore_barrier()
    pltpu.sync_copy(acc_S.at[pl.ds(sbase, STRIPE)],
                    accp_hbm.at[c, pl.ds(sbase, STRIPE)])

    @pl.when(s == 0)
    def _tail_out():
        pltpu.sync_copy(acc_S.at[pl.ds(NS * STRIPE, TAIL)],
                        accp_hbm.at[c, pl.ds(NS * STRIPE, TAIL)])


def _fin_body(h_ref, accp_ref, rsp_ref, out_ref):
    acc = accp_ref[0] + accp_ref[1]
    rs = jnp.sum(rsp_ref[...], axis=1, keepdims=True)
    hp = acc / rs
    hp = jnp.where(jnp.isnan(hp), jnp.zeros_like(hp), hp)
    hp = h_ref[...] + hp
    out_ref[...] = jnp.where(hp > 0, hp, jnp.exp(hp) - 1.0)


def _finalize(h, accp, rsp):
    return pl.pallas_call(
        _fin_body,
        grid=(NRB,),
        in_specs=[
            pl.BlockSpec((RB, D), lambda i: (i, 0)),
            pl.BlockSpec((NC, RB, D), lambda i: (0, i, 0)),
            pl.BlockSpec((RB, NW), lambda i: (i, 0)),
        ],
        out_specs=pl.BlockSpec((RB, D), lambda i: (i, 0)),
        out_shape=jax.ShapeDtypeStruct((N, D), jnp.float32),
    )(h, accp, rsp)


def kernel(node, edge, edge_feature, W, a):
    a8 = jnp.zeros((D, 8), jnp.float32)
    a8 = a8.at[:, 0].set(a[0, :D]).at[:, 1].set(a[0, D:2 * D])
    a3 = jnp.full((16,), a[0, 2 * D], jnp.float32)

    h, s8 = _matmul(node, W, a8)
    s8 = s8.reshape(N, 8)
    s1 = s8[:, 0]
    s2 = s8[:, 1]

    src = edge[:, 0]
    dst = edge[:, 1]
    ef = edge_feature[:, 0]
    zfull = jnp.zeros((STRIPE, D), jnp.float32)
    z1d = jnp.zeros((N,), jnp.float32)

    ee, accp, rsp = _edge_kernel(h, src, dst, ef, s1, s2, a3, zfull, z1d)
    out = _finalize(h, accp, rsp.T)
    return (out, ee.reshape(E, 1))


# speedup vs baseline: 7.7371x; 1.5243x over previous
import dataclasses
import functools

import jax
import jax.numpy as jnp
from jax import lax
from jax.experimental import pallas as pl
from jax.experimental.pallas import tpu as pltpu
from jax.experimental.pallas import tpu_sc as plsc

N = 10000
E = 320000
D = 128
ALPHA = 0.2

NC = 2
NS = 16
NW = NC * NS
EPW = E // NW
K = 64
NCHUNK = EPW // K
KT = EPW - NCHUNK * K
STRIPE = 624
TAIL = N - NS * STRIPE

RB = 1000
NRB = N // RB


def _mm_body(node_ref, w_ref, a8_ref, h_ref, s8_ref):
    hblk = jnp.dot(node_ref[...], w_ref[...],
                   preferred_element_type=jnp.float32,
                   precision=lax.Precision.HIGHEST)
    h_ref[...] = hblk
    s8_ref[0] = jnp.dot(hblk, a8_ref[...],
                        preferred_element_type=jnp.float32,
                        precision=lax.Precision.HIGHEST)


def _matmul(node, W, a8):
    return pl.pallas_call(
        _mm_body,
        grid=(NRB,),
        in_specs=[
            pl.BlockSpec((RB, D), lambda i: (i, 0)),
            pl.BlockSpec((D, D), lambda i: (0, 0)),
            pl.BlockSpec((D, 8), lambda i: (0, 0)),
        ],
        out_specs=[
            pl.BlockSpec((RB, D), lambda i: (i, 0)),
            pl.BlockSpec((1, RB, 8), lambda i: (i, 0, 0)),
        ],
        out_shape=[
            jax.ShapeDtypeStruct((N, D), jnp.float32),
            jax.ShapeDtypeStruct((NRB, RB, 8), jnp.float32),
        ],
    )(node, W, a8)


_sc_mesh = plsc.VectorSubcoreMesh(core_axis_name="c", subcore_axis_name="s")

_sc_params = pltpu.CompilerParams()
if "needs_layout_passes" in pltpu.CompilerParams.__dataclass_fields__:
    _sc_params = dataclasses.replace(_sc_params, needs_layout_passes=False)


@functools.partial(
    pl.kernel,
    mesh=_sc_mesh,
    compiler_params=_sc_params,
    out_type=[
        jax.ShapeDtypeStruct((E,), jnp.float32),
        jax.ShapeDtypeStruct((NC, N, D), jnp.float32),
        jax.ShapeDtypeStruct((NW, N), jnp.float32),
    ],
    scratch_types=[
        pltpu.VMEM((N,), jnp.float32),
        pltpu.VMEM((N,), jnp.float32),
        pltpu.VMEM((N,), jnp.float32),
        pltpu.VMEM((16,), jnp.float32),
        pltpu.VMEM((K,), jnp.int32),
        pltpu.VMEM((K,), jnp.int32),
        pltpu.VMEM((K,), jnp.float32),
        pltpu.VMEM((K,), jnp.float32),
        pltpu.VMEM((K,), jnp.int32),
        pltpu.VMEM((K, D), jnp.float32),
        pltpu.VMEM((K,), jnp.int32),
        pltpu.VMEM((K,), jnp.int32),
        pltpu.VMEM((K,), jnp.float32),
        pltpu.VMEM((K,), jnp.float32),
        pltpu.VMEM((K,), jnp.int32),
        pltpu.VMEM((K, D), jnp.float32),
        pltpu.VMEM((KT,), jnp.int32),
        pltpu.VMEM((KT,), jnp.int32),
        pltpu.SemaphoreType.DMA,
        pltpu.SemaphoreType.DMA,
        pltpu.SemaphoreType.DMA,
        pltpu.SemaphoreType.DMA,
        pltpu.SemaphoreType.DMA,
        pltpu.SemaphoreType.DMA,
        pltpu.SemaphoreType.DMA,
        pltpu.SemaphoreType.DMA,
        pltpu.VMEM_SHARED((N, D), jnp.float32),
    ],
)
def _edge_kernel(h_hbm, src_hbm, dst_hbm, ef_hbm, s1_hbm, s2_hbm, a3_hbm,
                 zfull_hbm, z1d_hbm,
                 ee_hbm, accp_hbm, rsp_hbm,
                 s1_v, s2_v, rs_v, a3_v,
                 src0, dst0, ef0, ee0, sidx0, rows0,
                 src1, dst1, ef1, ee1, sidx1, rows1, dst_t, sidx_t,
                 semi0, semg0, sems0, seme0,
                 semi1, semg1, sems1, seme1,
                 acc_S):
    c = lax.axis_index("c")
    s = lax.axis_index("s")
    wid = c * NS + s

    sbase = pl.multiple_of(s * STRIPE, 8)
    pltpu.sync_copy(zfull_hbm, acc_S.at[pl.ds(sbase, STRIPE)])

    @pl.when(s == 0)
    def _tail_init():
        pltpu.sync_copy(zfull_hbm.at[pl.ds(0, TAIL)],
                        acc_S.at[pl.ds(NS * STRIPE, TAIL)])
    pltpu.sync_copy(z1d_hbm, rs_v)
    pltpu.sync_copy(s1_hbm, s1_v)
    pltpu.sync_copy(s2_hbm, s2_v)
    pltpu.sync_copy(a3_hbm, a3_v)
    plsc.subcore_barrier()

    a3 = a3_v[...]
    base0 = wid * EPW

    sets = ((src0, dst0, ef0, ee0, sidx0, rows0, semi0, semg0, sems0, seme0),
            (src1, dst1, ef1, ee1, sidx1, rows1, semi1, semg1, sems1, seme1))

    def issue_idx(ci, S):
        src_v, dst_v, ef_v = S[0], S[1], S[2]
        base = base0 + ci * K
        pltpu.async_copy(src_hbm.at[pl.ds(base, K)], src_v, S[6])
        pltpu.async_copy(dst_hbm.at[pl.ds(base, K)], dst_v, S[6])
        pltpu.async_copy(ef_hbm.at[pl.ds(base, K)], ef_v, S[6])

    def wait_idx(S):
        pltpu.make_async_copy(src_hbm.at[pl.ds(0, K)], S[0], S[6]).wait()
        pltpu.make_async_copy(dst_hbm.at[pl.ds(0, K)], S[1], S[6]).wait()
        pltpu.make_async_copy(ef_hbm.at[pl.ds(0, K)], S[2], S[6]).wait()

    def body(ci, S, steady, prefetch):
        src_v, dst_v, ef_v, ee_v, sidx_v, rows_v = S[:6]
        semg, sems_, seme = S[7], S[8], S[9]
        other = sets[1] if S is sets[0] else sets[0]

        wait_idx(S)
        if steady:
            pltpu.make_async_copy(rows_v, acc_S.at[sidx_v], sems_).wait()
        pltpu.async_copy(h_hbm.at[dst_v], rows_v, semg)
        if prefetch:
            issue_idx(ci + 1, other)
        if steady:
            pltpu.make_async_copy(ee_v, ee_hbm.at[pl.ds(0, K)], seme).wait()

        @pl.loop(0, K, step=16)
        def _grp(g):
            src16 = src_v[pl.ds(g, 16)]
            dst16 = dst_v[pl.ds(g, 16)]
            ef16 = ef_v[pl.ds(g, 16)]
            v1 = plsc.load_gather(s1_v, [src16])
            v2 = plsc.load_gather(s2_v, [dst16])
            lg = v1 + v2 + ef16 * a3
            lr = jnp.where(lg >= 0, lg, ALPHA * lg)
            ee16 = jnp.exp(-lr)
            ee_v[pl.ds(g, 16)] = ee16
            plsc.addupdate_scatter(rs_v, [src16], ee16)
            sidx_v[pl.ds(g, 16)] = src16

        pltpu.make_async_copy(h_hbm.at[dst_v], rows_v, semg).wait()

        @pl.loop(0, K)
        def _edge(e):
            idx16 = lax.broadcast(e, (16,))
            spl = plsc.load_gather(ee_v, [idx16])
            for j in range(D // 16):
                rows_v[e, pl.ds(j * 16, 16)] = rows_v[e, pl.ds(j * 16, 16)] * spl

        pltpu.async_copy(rows_v, acc_S.at[sidx_v], sems_, add=True)
        pltpu.async_copy(ee_v, ee_hbm.at[pl.ds(base0 + ci * K, K)], seme)

    issue_idx(0, sets[0])
    body(0, sets[0], steady=False, prefetch=True)
    body(1, sets[1], steady=False, prefetch=True)

    @pl.loop(1, NCHUNK // 2 - 1)
    def _pair(i):
        body(2 * i, sets[0], steady=True, prefetch=True)
        body(2 * i + 1, sets[1], steady=True, prefetch=True)

    body(NCHUNK - 2, sets[0], steady=True, prefetch=True)
    body(NCHUNK - 1, sets[1], steady=True, prefetch=False)

    pltpu.make_async_copy(rows0, acc_S.at[sidx0], sems0).wait()
    pltpu.make_async_copy(ee0, ee_hbm.at[pl.ds(0, K)], seme0).wait()
    baset = base0 + NCHUNK * K
    pltpu.sync_copy(src_hbm.at[pl.ds(baset, KT)], src0.at[pl.ds(0, KT)])
    pltpu.sync_copy(dst_hbm.at[pl.ds(baset, KT)], dst_t)
    pltpu.sync_copy(ef_hbm.at[pl.ds(baset, KT)], ef0.at[pl.ds(0, KT)])
    pltpu.async_copy(h_hbm.at[dst_t], rows0.at[pl.ds(0, KT)], semg0).wait()
    src16 = src0[pl.ds(0, 16)]
    ef16 = ef0[pl.ds(0, 16)]
    v1 = plsc.load_gather(s1_v, [src16])
    v2 = plsc.load_gather(s2_v, [dst_t[...]])
    lg = v1 + v2 + ef16 * a3
    lr = jnp.where(lg >= 0, lg, ALPHA * lg)
    ee16 = jnp.exp(-lr)
    ee0[pl.ds(0, 16)] = ee16
    plsc.addupdate_scatter(rs_v, [src16], ee16)
    sidx_t[...] = src16

    @pl.loop(0, KT)
    def _tedge(e):
        idx16 = lax.broadcast(e, (16,))
        spl = plsc.load_gather(ee0, [idx16])
        for j in range(D // 16):
            rows0[e, pl.ds(j * 16, 16)] = rows0[e, pl.ds(j * 16, 16)] * spl

    pltpu.sync_copy(rows0.at[pl.ds(0, KT)], acc_S.at[sidx_t], add=True)
    pltpu.sync_copy(ee0.at[pl.ds(0, KT)], ee_hbm.at[pl.ds(baset, KT)])

    pltpu.make_async_copy(rows1, acc_S.at[sidx1], sems1).wait()
    pltpu.make_async_copy(ee1, ee_hbm.at[pl.ds(0, K)], seme1).wait()

    pltpu.sync_copy(rs_v, rsp_hbm.at[wid])
    plsc.subcore_barrier()
    pltpu.sync_copy(acc_S.at[pl.ds(sbase, STRIPE)],
                    accp_hbm.at[c, pl.ds(sbase, STRIPE)])

    @pl.when(s == 0)
    def _tail_out():
        pltpu.sync_copy(acc_S.at[pl.ds(NS * STRIPE, TAIL)],
                        accp_hbm.at[c, pl.ds(NS * STRIPE, TAIL)])


def _fin_body(h_ref, accp_ref, rsp_ref, out_ref):
    acc = accp_ref[0] + accp_ref[1]
    rs = jnp.sum(rsp_ref[...], axis=1, keepdims=True)
    hp = acc / rs
    hp = jnp.where(jnp.isnan(hp), jnp.zeros_like(hp), hp)
    hp = h_ref[...] + hp
    out_ref[...] = jnp.where(hp > 0, hp, jnp.exp(hp) - 1.0)


def _finalize(h, accp, rsp):
    return pl.pallas_call(
        _fin_body,
        grid=(NRB,),
        in_specs=[
            pl.BlockSpec((RB, D), lambda i: (i, 0)),
            pl.BlockSpec((NC, RB, D), lambda i: (0, i, 0)),
            pl.BlockSpec((RB, NW), lambda i: (i, 0)),
        ],
        out_specs=pl.BlockSpec((RB, D), lambda i: (i, 0)),
        out_shape=jax.ShapeDtypeStruct((N, D), jnp.float32),
    )(h, accp, rsp)


def kernel(node, edge, edge_feature, W, a):
    a8 = jnp.zeros((D, 8), jnp.float32)
    a8 = a8.at[:, 0].set(a[0, :D]).at[:, 1].set(a[0, D:2 * D])
    a3 = jnp.full((16,), a[0, 2 * D], jnp.float32)

    h, s8 = _matmul(node, W, a8)
    s8 = s8.reshape(N, 8)
    s1 = s8[:, 0]
    s2 = s8[:, 1]

    src = edge[:, 0]
    dst = edge[:, 1]
    ef = edge_feature[:, 0]
    zfull = jnp.zeros((STRIPE, D), jnp.float32)
    z1d = jnp.zeros((N,), jnp.float32)

    ee, accp, rsp = _edge_kernel(h, src, dst, ef, s1, s2, a3, zfull, z1d)
    out = _finalize(h, accp, rsp.T)
    return (out, ee.reshape(E, 1))


# speedup vs baseline: 8.7916x; 1.1363x over previous
import dataclasses
import functools

import jax
import jax.numpy as jnp
from jax import lax
from jax.experimental import pallas as pl
from jax.experimental.pallas import tpu as pltpu
from jax.experimental.pallas import tpu_sc as plsc

N = 10000
E = 320000
D = 128
ALPHA = 0.2

NC = 2
NS = 16
NW = NC * NS
EPW = E // NW
K = 64
NCHUNK = EPW // K
KT = EPW - NCHUNK * K
STRIPE = 624
TAIL = N - NS * STRIPE

RB = 1000
NRB = N // RB


def _mm_body(node_ref, w_ref, a8_ref, h_ref, s8_ref):
    hblk = jnp.dot(node_ref[...], w_ref[...],
                   preferred_element_type=jnp.float32,
                   precision=lax.Precision.HIGHEST)
    h_ref[...] = hblk
    s8_ref[0] = jnp.dot(hblk, a8_ref[...],
                        preferred_element_type=jnp.float32,
                        precision=lax.Precision.HIGHEST)


def _matmul(node, W, a8):
    return pl.pallas_call(
        _mm_body,
        grid=(NRB,),
        in_specs=[
            pl.BlockSpec((RB, D), lambda i: (i, 0)),
            pl.BlockSpec((D, D), lambda i: (0, 0)),
            pl.BlockSpec((D, 8), lambda i: (0, 0)),
        ],
        out_specs=[
            pl.BlockSpec((RB, D), lambda i: (i, 0)),
            pl.BlockSpec((1, RB, 8), lambda i: (i, 0, 0)),
        ],
        out_shape=[
            jax.ShapeDtypeStruct((N, D), jnp.float32),
            jax.ShapeDtypeStruct((NRB, RB, 8), jnp.float32),
        ],
    )(node, W, a8)


_sc_mesh = plsc.VectorSubcoreMesh(core_axis_name="c", subcore_axis_name="s")

_sc_params = pltpu.CompilerParams()
if "needs_layout_passes" in pltpu.CompilerParams.__dataclass_fields__:
    _sc_params = dataclasses.replace(_sc_params, needs_layout_passes=False)


@functools.partial(
    pl.kernel,
    mesh=_sc_mesh,
    compiler_params=_sc_params,
    out_type=[
        jax.ShapeDtypeStruct((E,), jnp.float32),
        jax.ShapeDtypeStruct((NC, N, D), jnp.float32),
        jax.ShapeDtypeStruct((NW, N), jnp.float32),
    ],
    scratch_types=[
        pltpu.VMEM((N,), jnp.float32),
        pltpu.VMEM((N,), jnp.float32),
        pltpu.VMEM((N,), jnp.float32),
        pltpu.VMEM((16,), jnp.float32),
        pltpu.VMEM((K,), jnp.int32),
        pltpu.VMEM((K,), jnp.int32),
        pltpu.VMEM((K,), jnp.float32),
        pltpu.VMEM((K,), jnp.float32),
        pltpu.VMEM((K,), jnp.int32),
        pltpu.VMEM((K, D), jnp.float32),
        pltpu.VMEM((K,), jnp.int32),
        pltpu.VMEM((K,), jnp.int32),
        pltpu.VMEM((K,), jnp.float32),
        pltpu.VMEM((K,), jnp.float32),
        pltpu.VMEM((K,), jnp.int32),
        pltpu.VMEM((K, D), jnp.float32),
        pltpu.VMEM((KT,), jnp.int32),
        pltpu.VMEM((KT,), jnp.int32),
        pltpu.SemaphoreType.DMA,
        pltpu.SemaphoreType.DMA,
        pltpu.SemaphoreType.DMA,
        pltpu.SemaphoreType.DMA,
        pltpu.SemaphoreType.DMA,
        pltpu.SemaphoreType.DMA,
        pltpu.SemaphoreType.DMA,
        pltpu.SemaphoreType.DMA,
        pltpu.VMEM_SHARED((N, D), jnp.float32),
    ],
)
def _edge_kernel(h_hbm, src_hbm, dst_hbm, ef_hbm, s1_hbm, s2_hbm, a3_hbm,
                 zfull_hbm, z1d_hbm,
                 ee_hbm, accp_hbm, rsp_hbm,
                 s1_v, s2_v, rs_v, a3_v,
                 src0, dst0, ef0, ee0, sidx0, rows0,
                 src1, dst1, ef1, ee1, sidx1, rows1, dst_t, sidx_t,
                 semi0, semg0, sems0, seme0,
                 semi1, semg1, sems1, seme1,
                 acc_S):
    c = lax.axis_index("c")
    s = lax.axis_index("s")
    wid = c * NS + s

    sbase = pl.multiple_of(s * STRIPE, 8)
    pltpu.sync_copy(zfull_hbm, acc_S.at[pl.ds(sbase, STRIPE)])

    @pl.when(s == 0)
    def _tail_init():
        pltpu.sync_copy(zfull_hbm.at[pl.ds(0, TAIL)],
                        acc_S.at[pl.ds(NS * STRIPE, TAIL)])
    pltpu.sync_copy(z1d_hbm, rs_v)
    pltpu.sync_copy(s1_hbm, s1_v)
    pltpu.sync_copy(s2_hbm, s2_v)
    pltpu.sync_copy(a3_hbm, a3_v)
    plsc.subcore_barrier()

    a3 = a3_v[...]
    base0 = wid * EPW

    sets = ((src0, dst0, ef0, ee0, sidx0, rows0, semi0, semg0, sems0, seme0),
            (src1, dst1, ef1, ee1, sidx1, rows1, semi1, semg1, sems1, seme1))

    def issue_idx(ci, S):
        src_v, dst_v, ef_v = S[0], S[1], S[2]
        base = base0 + ci * K
        pltpu.async_copy(src_hbm.at[pl.ds(base, K)], src_v, S[6])
        pltpu.async_copy(dst_hbm.at[pl.ds(base, K)], dst_v, S[6])
        pltpu.async_copy(ef_hbm.at[pl.ds(base, K)], ef_v, S[6])

    def wait_idx(S):
        pltpu.make_async_copy(src_hbm.at[pl.ds(0, K)], S[0], S[6]).wait()
        pltpu.make_async_copy(dst_hbm.at[pl.ds(0, K)], S[1], S[6]).wait()
        pltpu.make_async_copy(ef_hbm.at[pl.ds(0, K)], S[2], S[6]).wait()

    def body(ci, S, steady, prefetch):
        src_v, dst_v, ef_v, ee_v, sidx_v, rows_v = S[:6]
        semg, sems_, seme = S[7], S[8], S[9]
        other = sets[1] if S is sets[0] else sets[0]

        wait_idx(S)
        if steady:
            pltpu.make_async_copy(rows_v, acc_S.at[sidx_v], sems_).wait()
        pltpu.async_copy(h_hbm.at[dst_v], rows_v, semg)
        if prefetch:
            issue_idx(ci + 1, other)
        if steady:
            pltpu.make_async_copy(ee_v, ee_hbm.at[pl.ds(0, K)], seme).wait()

        @pl.loop(0, K, step=16)
        def _grp(g):
            src16 = src_v[pl.ds(g, 16)]
            dst16 = dst_v[pl.ds(g, 16)]
            ef16 = ef_v[pl.ds(g, 16)]
            v1 = plsc.load_gather(s1_v, [src16])
            v2 = plsc.load_gather(s2_v, [dst16])
            lg = v1 + v2 + ef16 * a3
            lr = jnp.where(lg >= 0, lg, ALPHA * lg)
            ee16 = jnp.exp(-lr)
            ee_v[pl.ds(g, 16)] = ee16
            plsc.addupdate_scatter(rs_v, [src16], ee16)
            sidx_v[pl.ds(g, 16)] = src16

        pltpu.make_async_copy(h_hbm.at[dst_v], rows_v, semg).wait()

        @plsc.parallel_loop(0, K, 1, unroll=4)
        def _edge(e):
            idx16 = lax.broadcast(e, (16,))
            spl = plsc.load_gather(ee_v, [idx16])
            for j in range(D // 16):
                rows_v[e, pl.ds(j * 16, 16)] = rows_v[e, pl.ds(j * 16, 16)] * spl

        pltpu.async_copy(rows_v, acc_S.at[sidx_v], sems_, add=True)
        pltpu.async_copy(ee_v, ee_hbm.at[pl.ds(base0 + ci * K, K)], seme)

    issue_idx(0, sets[0])
    body(0, sets[0], steady=False, prefetch=True)
    body(1, sets[1], steady=False, prefetch=True)

    @pl.loop(1, NCHUNK // 2 - 1)
    def _pair(i):
        body(2 * i, sets[0], steady=True, prefetch=True)
        body(2 * i + 1, sets[1], steady=True, prefetch=True)

    body(NCHUNK - 2, sets[0], steady=True, prefetch=True)
    body(NCHUNK - 1, sets[1], steady=True, prefetch=False)

    pltpu.make_async_copy(rows0, acc_S.at[sidx0], sems0).wait()
    pltpu.make_async_copy(ee0, ee_hbm.at[pl.ds(0, K)], seme0).wait()
    baset = base0 + NCHUNK * K
    pltpu.sync_copy(src_hbm.at[pl.ds(baset, KT)], src0.at[pl.ds(0, KT)])
    pltpu.sync_copy(dst_hbm.at[pl.ds(baset, KT)], dst_t)
    pltpu.sync_copy(ef_hbm.at[pl.ds(baset, KT)], ef0.at[pl.ds(0, KT)])
    pltpu.async_copy(h_hbm.at[dst_t], rows0.at[pl.ds(0, KT)], semg0).wait()
    src16 = src0[pl.ds(0, 16)]
    ef16 = ef0[pl.ds(0, 16)]
    v1 = plsc.load_gather(s1_v, [src16])
    v2 = plsc.load_gather(s2_v, [dst_t[...]])
    lg = v1 + v2 + ef16 * a3
    lr = jnp.where(lg >= 0, lg, ALPHA * lg)
    ee16 = jnp.exp(-lr)
    ee0[pl.ds(0, 16)] = ee16
    plsc.addupdate_scatter(rs_v, [src16], ee16)
    sidx_t[...] = src16

    @pl.loop(0, KT)
    def _tedge(e):
        idx16 = lax.broadcast(e, (16,))
        spl = plsc.load_gather(ee0, [idx16])
        for j in range(D // 16):
            rows0[e, pl.ds(j * 16, 16)] = rows0[e, pl.ds(j * 16, 16)] * spl

    pltpu.sync_copy(rows0.at[pl.ds(0, KT)], acc_S.at[sidx_t], add=True)
    pltpu.sync_copy(ee0.at[pl.ds(0, KT)], ee_hbm.at[pl.ds(baset, KT)])

    pltpu.make_async_copy(rows1, acc_S.at[sidx1], sems1).wait()
    pltpu.make_async_copy(ee1, ee_hbm.at[pl.ds(0, K)], seme1).wait()

    pltpu.sync_copy(rs_v, rsp_hbm.at[wid])
    plsc.subcore_barrier()
    pltpu.sync_copy(acc_S.at[pl.ds(sbase, STRIPE)],
                    accp_hbm.at[c, pl.ds(sbase, STRIPE)])

    @pl.when(s == 0)
    def _tail_out():
        pltpu.sync_copy(acc_S.at[pl.ds(NS * STRIPE, TAIL)],
                        accp_hbm.at[c, pl.ds(NS * STRIPE, TAIL)])


def _fin_body(h_ref, accp_ref, rsp_ref, out_ref):
    acc = accp_ref[0] + accp_ref[1]
    rs = jnp.sum(rsp_ref[...], axis=1, keepdims=True)
    hp = acc / rs
    hp = jnp.where(jnp.isnan(hp), jnp.zeros_like(hp), hp)
    hp = h_ref[...] + hp
    out_ref[...] = jnp.where(hp > 0, hp, jnp.exp(hp) - 1.0)


def _finalize(h, accp, rsp):
    return pl.pallas_call(
        _fin_body,
        grid=(NRB,),
        in_specs=[
            pl.BlockSpec((RB, D), lambda i: (i, 0)),
            pl.BlockSpec((NC, RB, D), lambda i: (0, i, 0)),
            pl.BlockSpec((RB, NW), lambda i: (i, 0)),
        ],
        out_specs=pl.BlockSpec((RB, D), lambda i: (i, 0)),
        out_shape=jax.ShapeDtypeStruct((N, D), jnp.float32),
    )(h, accp, rsp)


def kernel(node, edge, edge_feature, W, a):
    a8 = jnp.zeros((D, 8), jnp.float32)
    a8 = a8.at[:, 0].set(a[0, :D]).at[:, 1].set(a[0, D:2 * D])
    a3 = jnp.full((16,), a[0, 2 * D], jnp.float32)

    h, s8 = _matmul(node, W, a8)
    s8 = s8.reshape(N, 8)
    s1 = s8[:, 0]
    s2 = s8[:, 1]

    src = edge[:, 0]
    dst = edge[:, 1]
    ef = edge_feature[:, 0]
    zfull = jnp.zeros((STRIPE, D), jnp.float32)
    z1d = jnp.zeros((N,), jnp.float32)

    ee, accp, rsp = _edge_kernel(h, src, dst, ef, s1, s2, a3, zfull, z1d)
    out = _finalize(h, accp, rsp.T)
    return (out, ee.reshape(E, 1))


# speedup vs baseline: 10.5514x; 1.2002x over previous
import dataclasses
import functools

import jax
import jax.numpy as jnp
from jax import lax
from jax.experimental import pallas as pl
from jax.experimental.pallas import tpu as pltpu
from jax.experimental.pallas import tpu_sc as plsc

N = 10000
E = 320000
D = 128
ALPHA = 0.2

NC = 2
NS = 16
NW = NC * NS
EPW = E // NW
K = 48
NCHUNK = EPW // K
KT = EPW - NCHUNK * K
STRIPE = 624
TAIL = N - NS * STRIPE

RB = 1000
NRB = N // RB


def _mm_body(node_ref, w_ref, a8_ref, h_ref, s8_ref):
    hblk = jnp.dot(node_ref[...], w_ref[...],
                   preferred_element_type=jnp.float32,
                   precision=lax.Precision.HIGHEST)
    h_ref[...] = hblk
    s8_ref[0] = jnp.dot(hblk, a8_ref[...],
                        preferred_element_type=jnp.float32,
                        precision=lax.Precision.HIGHEST)


def _matmul(node, W, a8):
    return pl.pallas_call(
        _mm_body,
        grid=(NRB,),
        in_specs=[
            pl.BlockSpec((RB, D), lambda i: (i, 0)),
            pl.BlockSpec((D, D), lambda i: (0, 0)),
            pl.BlockSpec((D, 8), lambda i: (0, 0)),
        ],
        out_specs=[
            pl.BlockSpec((RB, D), lambda i: (i, 0)),
            pl.BlockSpec((1, RB, 8), lambda i: (i, 0, 0)),
        ],
        out_shape=[
            jax.ShapeDtypeStruct((N, D), jnp.float32),
            jax.ShapeDtypeStruct((NRB, RB, 8), jnp.float32),
        ],
    )(node, W, a8)


_sc_mesh = plsc.VectorSubcoreMesh(core_axis_name="c", subcore_axis_name="s")

_sc_params = pltpu.CompilerParams()
if "needs_layout_passes" in pltpu.CompilerParams.__dataclass_fields__:
    _sc_params = dataclasses.replace(_sc_params, needs_layout_passes=False)


@functools.partial(
    pl.kernel,
    mesh=_sc_mesh,
    compiler_params=_sc_params,
    out_type=[
        jax.ShapeDtypeStruct((E,), jnp.float32),
        jax.ShapeDtypeStruct((NC, N, D), jnp.float32),
        jax.ShapeDtypeStruct((NW, N), jnp.float32),
    ],
    scratch_types=[
        pltpu.VMEM((N,), jnp.float32),
        pltpu.VMEM((N,), jnp.float32),
        pltpu.VMEM((N,), jnp.float32),
        pltpu.VMEM((16,), jnp.float32),
        pltpu.VMEM((K,), jnp.int32),
        pltpu.VMEM((K,), jnp.int32),
        pltpu.VMEM((K,), jnp.float32),
        pltpu.VMEM((K,), jnp.float32),
        pltpu.VMEM((K,), jnp.int32),
        pltpu.VMEM((K, D), jnp.float32),
        pltpu.VMEM((K,), jnp.int32),
        pltpu.VMEM((K,), jnp.int32),
        pltpu.VMEM((K,), jnp.float32),
        pltpu.VMEM((K,), jnp.float32),
        pltpu.VMEM((K,), jnp.int32),
        pltpu.VMEM((K, D), jnp.float32),
        pltpu.VMEM((K,), jnp.int32),
        pltpu.VMEM((K,), jnp.int32),
        pltpu.VMEM((K,), jnp.float32),
        pltpu.VMEM((K,), jnp.float32),
        pltpu.VMEM((K,), jnp.int32),
        pltpu.VMEM((K, D), jnp.float32),
        pltpu.VMEM((KT,), jnp.int32),
        pltpu.VMEM((KT,), jnp.int32),
        pltpu.SemaphoreType.DMA,
        pltpu.SemaphoreType.DMA,
        pltpu.SemaphoreType.DMA,
        pltpu.SemaphoreType.DMA,
        pltpu.SemaphoreType.DMA,
        pltpu.SemaphoreType.DMA,
        pltpu.SemaphoreType.DMA,
        pltpu.SemaphoreType.DMA,
        pltpu.SemaphoreType.DMA,
        pltpu.SemaphoreType.DMA,
        pltpu.SemaphoreType.DMA,
        pltpu.SemaphoreType.DMA,
        pltpu.VMEM_SHARED((N, D), jnp.float32),
    ],
)
def _edge_kernel(h_hbm, src_hbm, dst_hbm, ef_hbm, s1_hbm, s2_hbm, a3_hbm,
                 zfull_hbm, z1d_hbm,
                 ee_hbm, accp_hbm, rsp_hbm,
                 s1_v, s2_v, rs_v, a3_v,
                 src0, dst0, ef0, ee0, sidx0, rows0,
                 src1, dst1, ef1, ee1, sidx1, rows1,
                 src2, dst2, ef2, ee2, sidx2, rows2, dst_t, sidx_t,
                 semi0, semg0, sems0, seme0,
                 semi1, semg1, sems1, seme1,
                 semi2, semg2, sems2, seme2,
                 acc_S):
    c = lax.axis_index("c")
    s = lax.axis_index("s")
    wid = c * NS + s

    sbase = pl.multiple_of(s * STRIPE, 8)
    pltpu.sync_copy(zfull_hbm, acc_S.at[pl.ds(sbase, STRIPE)])

    @pl.when(s == 0)
    def _tail_init():
        pltpu.sync_copy(zfull_hbm.at[pl.ds(0, TAIL)],
                        acc_S.at[pl.ds(NS * STRIPE, TAIL)])
    pltpu.sync_copy(z1d_hbm, rs_v)
    pltpu.sync_copy(s1_hbm, s1_v)
    pltpu.sync_copy(s2_hbm, s2_v)
    pltpu.sync_copy(a3_hbm, a3_v)
    plsc.subcore_barrier()

    a3 = a3_v[...]
    base0 = wid * EPW

    sets = ((src0, dst0, ef0, ee0, sidx0, rows0, semi0, semg0, sems0, seme0),
            (src1, dst1, ef1, ee1, sidx1, rows1, semi1, semg1, sems1, seme1),
            (src2, dst2, ef2, ee2, sidx2, rows2, semi2, semg2, sems2, seme2))

    def issue_idx(ci, S):
        base = base0 + ci * K
        pltpu.async_copy(src_hbm.at[pl.ds(base, K)], S[0], S[6])
        pltpu.async_copy(dst_hbm.at[pl.ds(base, K)], S[1], S[6])
        pltpu.async_copy(ef_hbm.at[pl.ds(base, K)], S[2], S[6])

    def wait_idx(S):
        pltpu.make_async_copy(src_hbm.at[pl.ds(0, K)], S[0], S[6]).wait()
        pltpu.make_async_copy(dst_hbm.at[pl.ds(0, K)], S[1], S[6]).wait()
        pltpu.make_async_copy(ef_hbm.at[pl.ds(0, K)], S[2], S[6]).wait()

    def wait_scatter(S):
        pltpu.make_async_copy(S[5], acc_S.at[S[4]], S[8]).wait()

    def wait_ee_wb(S):
        pltpu.make_async_copy(S[3], ee_hbm.at[pl.ds(0, K)], S[9]).wait()

    def body(ci, si, wait_ew, wait_sc, gather_next, idx_next2):
        S = sets[si]
        Sn = sets[(si + 1) % 3]
        src_v, dst_v, ef_v, ee_v, sidx_v, rows_v = S[:6]
        semg, sems_, seme = S[7], S[8], S[9]

        if wait_ew:
            wait_ee_wb(S)

        @pl.loop(0, K, step=16)
        def _grp(g):
            src16 = src_v[pl.ds(g, 16)]
            dst16 = dst_v[pl.ds(g, 16)]
            ef16 = ef_v[pl.ds(g, 16)]
            v1 = plsc.load_gather(s1_v, [src16])
            v2 = plsc.load_gather(s2_v, [dst16])
            lg = v1 + v2 + ef16 * a3
            lr = jnp.where(lg >= 0, lg, ALPHA * lg)
            ee16 = jnp.exp(-lr)
            ee_v[pl.ds(g, 16)] = ee16
            plsc.addupdate_scatter(rs_v, [src16], ee16)
            sidx_v[pl.ds(g, 16)] = src16

        if wait_sc:
            wait_scatter(Sn)
        if gather_next:
            wait_idx(Sn)
            pltpu.async_copy(h_hbm.at[Sn[1]], Sn[5], Sn[7])
        pltpu.make_async_copy(h_hbm.at[dst_v], rows_v, semg).wait()
        if idx_next2:
            issue_idx(ci + 2, sets[(si + 2) % 3])

        @plsc.parallel_loop(0, K, 1, unroll=4)
        def _edge(e):
            idx16 = lax.broadcast(e, (16,))
            spl = plsc.load_gather(ee_v, [idx16])
            for j in range(D // 16):
                rows_v[e, pl.ds(j * 16, 16)] = rows_v[e, pl.ds(j * 16, 16)] * spl

        pltpu.async_copy(rows_v, acc_S.at[sidx_v], sems_, add=True)
        pltpu.async_copy(ee_v, ee_hbm.at[pl.ds(base0 + ci * K, K)], seme)

    issue_idx(0, sets[0])
    issue_idx(1, sets[1])
    wait_idx(sets[0])
    pltpu.async_copy(h_hbm.at[sets[0][1]], sets[0][5], sets[0][7])

    body(0, 0, False, False, True, True)
    body(1, 1, False, False, True, True)
    body(2, 2, False, True, True, True)
    body(3, 0, True, True, True, True)
    body(4, 1, True, True, True, True)

    @pl.loop(0, (NCHUNK - 5 - 2) // 3)
    def _tri(i):
        ci = 5 + 3 * i
        body(ci, 2, True, True, True, True)
        body(ci + 1, 0, True, True, True, True)
        body(ci + 2, 1, True, True, True, True)

    body(NCHUNK - 2, 2, True, True, True, False)
    body(NCHUNK - 1, 0, True, True, False, False)

    St = sets[1]
    wait_ee_wb(St)
    baset = base0 + NCHUNK * K
    pltpu.sync_copy(src_hbm.at[pl.ds(baset, KT)], St[0].at[pl.ds(0, KT)])
    pltpu.sync_copy(dst_hbm.at[pl.ds(baset, KT)], dst_t)
    pltpu.sync_copy(ef_hbm.at[pl.ds(baset, KT)], St[2].at[pl.ds(0, KT)])
    pltpu.async_copy(h_hbm.at[dst_t], St[5].at[pl.ds(0, KT)], St[7]).wait()
    src16 = St[0][pl.ds(0, 16)]
    ef16 = St[2][pl.ds(0, 16)]
    v1 = plsc.load_gather(s1_v, [src16])
    v2 = plsc.load_gather(s2_v, [dst_t[...]])
    lg = v1 + v2 + ef16 * a3
    lr = jnp.where(lg >= 0, lg, ALPHA * lg)
    ee16 = jnp.exp(-lr)
    St[3][pl.ds(0, 16)] = ee16
    plsc.addupdate_scatter(rs_v, [src16], ee16)
    sidx_t[...] = src16

    @pl.loop(0, KT)
    def _tedge(e):
        idx16 = lax.broadcast(e, (16,))
        spl = plsc.load_gather(St[3], [idx16])
        for j in range(D // 16):
            St[5][e, pl.ds(j * 16, 16)] = St[5][e, pl.ds(j * 16, 16)] * spl

    pltpu.sync_copy(St[5].at[pl.ds(0, KT)], acc_S.at[sidx_t], add=True)
    pltpu.sync_copy(St[3].at[pl.ds(0, KT)], ee_hbm.at[pl.ds(baset, KT)])

    wait_scatter(sets[(NCHUNK - 2) % 3])
    wait_ee_wb(sets[(NCHUNK - 2) % 3])
    wait_scatter(sets[(NCHUNK - 1) % 3])
    wait_ee_wb(sets[(NCHUNK - 1) % 3])

    pltpu.sync_copy(rs_v, rsp_hbm.at[wid])
    plsc.subcore_barrier()
    pltpu.sync_copy(acc_S.at[pl.ds(sbase, STRIPE)],
                    accp_hbm.at[c, pl.ds(sbase, STRIPE)])

    @pl.when(s == 0)
    def _tail_out():
        pltpu.sync_copy(acc_S.at[pl.ds(NS * STRIPE, TAIL)],
                        accp_hbm.at[c, pl.ds(NS * STRIPE, TAIL)])


def _fin_body(h_ref, accp_ref, rsp_ref, out_ref):
    acc = accp_ref[0] + accp_ref[1]
    rs = jnp.sum(rsp_ref[...], axis=1, keepdims=True)
    hp = acc / rs
    hp = jnp.where(jnp.isnan(hp), jnp.zeros_like(hp), hp)
    hp = h_ref[...] + hp
    out_ref[...] = jnp.where(hp > 0, hp, jnp.exp(hp) - 1.0)


def _finalize(h, accp, rsp):
    return pl.pallas_call(
        _fin_body,
        grid=(NRB,),
        in_specs=[
            pl.BlockSpec((RB, D), lambda i: (i, 0)),
            pl.BlockSpec((NC, RB, D), lambda i: (0, i, 0)),
            pl.BlockSpec((RB, NW), lambda i: (i, 0)),
        ],
        out_specs=pl.BlockSpec((RB, D), lambda i: (i, 0)),
        out_shape=jax.ShapeDtypeStruct((N, D), jnp.float32),
    )(h, accp, rsp)


def kernel(node, edge, edge_feature, W, a):
    a8 = jnp.zeros((D, 8), jnp.float32)
    a8 = a8.at[:, 0].set(a[0, :D]).at[:, 1].set(a[0, D:2 * D])
    a3 = jnp.full((16,), a[0, 2 * D], jnp.float32)

    h, s8 = _matmul(node, W, a8)
    s8 = s8.reshape(N, 8)
    s1 = s8[:, 0]
    s2 = s8[:, 1]

    src = edge[:, 0]
    dst = edge[:, 1]
    ef = edge_feature[:, 0]
    zfull = jnp.zeros((STRIPE, D), jnp.float32)
    z1d = jnp.zeros((N,), jnp.float32)

    ee, accp, rsp = _edge_kernel(h, src, dst, ef, s1, s2, a3, zfull, z1d)
    out = _finalize(h, accp, rsp.T)
    return (out, ee.reshape(E, 1))
